# Initial kernel scaffold; baseline (speedup 1.0000x reference)
import jax
import jax.numpy as jnp
from jax.experimental import pallas as pl


def kernel(x, router_w, fc_w, proj_w):
    raise NotImplementedError("write your pallas kernel here")


# speedup vs baseline: 1.3074x; 1.3074x over previous
import jax
import jax.numpy as jnp
from jax.experimental import pallas as pl
from jax.experimental.pallas import tpu as pltpu

DIM = 1024
HID = 2048
E = 8
T = 2048
TT = 512


def _router(xb, rw):
    logits = jnp.dot(xb, rw.T, preferred_element_type=jnp.float32)
    idx = jax.lax.broadcasted_iota(jnp.int32, logits.shape, 1)
    m1 = jnp.max(logits, axis=1, keepdims=True)
    i1 = jnp.min(jnp.where(logits == m1, idx, E), axis=1, keepdims=True)
    masked = jnp.where(idx == i1, -jnp.inf, logits)
    m2 = jnp.max(masked, axis=1, keepdims=True)
    i2 = jnp.min(jnp.where(masked == m2, idx, E), axis=1, keepdims=True)
    z = jnp.exp(logits - m1)
    denom = jnp.sum(z, axis=1, keepdims=True)
    p1 = 1.0 / denom
    p2 = jnp.exp(m2 - m1) / denom
    s = p1 + p2 + 1e-8
    w1 = p1 / s
    w2 = p2 / s
    return jnp.where(idx == i1, w1, 0.0) + jnp.where(idx == i2, w2, 0.0)


def _moe_kernel(x_ref, rw_ref, fc_ref, pj_ref, out_ref, ew_ref):
    e = pl.program_id(1)
    xb = x_ref[...]

    @pl.when(e == 0)
    def _():
        ew_ref[...] = _router(xb, rw_ref[...])
        out_ref[...] = jnp.zeros_like(out_ref)

    h = jnp.dot(xb, fc_ref[0].T, preferred_element_type=jnp.float32)
    h = jnp.square(jnp.maximum(h, 0.0))
    y = jnp.dot(h, pj_ref[0].T, preferred_element_type=jnp.float32)
    ew = ew_ref[...]
    lane = jax.lax.broadcasted_iota(jnp.int32, ew.shape, 1)
    w = jnp.sum(jnp.where(lane == e, ew, 0.0), axis=1, keepdims=True)
    out_ref[...] += y * w


def kernel(x, router_w, fc_w, proj_w):
    bsz, seq_len, dim = x.shape
    x_flat = x.reshape(-1, dim)
    grid = (T // TT, E)
    out = pl.pallas_call(
        _moe_kernel,
        grid=grid,
        in_specs=[
            pl.BlockSpec((TT, DIM), lambda t, e: (t, 0)),
            pl.BlockSpec((E, DIM), lambda t, e: (0, 0)),
            pl.BlockSpec((1, HID, DIM), lambda t, e: (e, 0, 0)),
            pl.BlockSpec((1, DIM, HID), lambda t, e: (e, 0, 0)),
        ],
        out_specs=pl.BlockSpec((TT, DIM), lambda t, e: (t, 0)),
        out_shape=jax.ShapeDtypeStruct((T, DIM), jnp.float32),
        scratch_shapes=[pltpu.VMEM((TT, E), jnp.float32)],
        compiler_params=pltpu.CompilerParams(
            dimension_semantics=("arbitrary", "arbitrary"),
        ),
    )(x_flat, router_w, fc_w, proj_w)
    return out.reshape(bsz, seq_len, dim), jnp.float32(0.0)
